# baseline (speedup 1.0000x reference)
import jax
import jax.numpy as jnp
import numpy as np
from jax.experimental import pallas as pl
from jax.experimental.pallas import tpu as pltpu

_CHANNEL_NUM = 64
_REGIONS = [list(range(i * 8, (i + 1) * 8)) for i in range(8)]


def _channel_perm_build() -> tuple:
    perm = np.arange(_CHANNEL_NUM, dtype=np.int32)
    key = jax.random.key(42)
    for r, region in enumerate(_REGIONS):
        idx = np.array([c for c in region if c < _CHANNEL_NUM], dtype=np.int32)
        if len(idx) < 2:
            continue
        shuffled = jax.random.permutation(
            jax.random.fold_in(key, r), jnp.asarray(idx)
        )
        perm[idx] = np.asarray(shuffled)
    return tuple(int(x) for x in perm)


_PERM = _channel_perm_build()


def _channel_perm() -> tuple:
    return _PERM


def _copy_kernel(perm_ref, in_ref, out_ref):
    del perm_ref
    out_ref[...] = in_ref[...]


def kernel(data_tensor, domain_labels, aux_labels):
    del domain_labels, aux_labels
    perm = _channel_perm()
    b, c, one, t = data_tensor.shape
    perm_arr = jnp.asarray(np.array(perm, dtype=np.int32))

    grid_spec = pltpu.PrefetchScalarGridSpec(
        num_scalar_prefetch=1,
        grid=(c,),
        in_specs=[
            pl.BlockSpec(
                (b, 1, one, t),
                lambda ci, perm_ref: (0, perm_ref[ci], 0, 0),
            )
        ],
        out_specs=pl.BlockSpec(
            (b, 1, one, t), lambda ci, perm_ref: (0, ci, 0, 0)
        ),
    )
    out = pl.pallas_call(
        _copy_kernel,
        grid_spec=grid_spec,
        out_shape=jax.ShapeDtypeStruct((b, c, one, t), data_tensor.dtype),
    )(perm_arr, data_tensor)
    return out
